# baseline (speedup 1.0000x reference)
import jax
import jax.numpy as jnp
from jax import lax
from jax.experimental import pallas as pl
from jax.experimental.pallas import tpu as pltpu
from jax.experimental.pallas import tpu_sc as plsc

B = 8
S = 512
H = 256
ML = 2048
NC, NS = 2, 16
NW = NC * NS
WPB = NW // B
FPW = ML // WPB
CHUNK = 64
NCHUNK = FPW // CHUNK
NBUF = 4
LANES = 16
VPC = CHUNK // LANES
STRIDE = 32
COMB = FPW // STRIDE
NCOMB = CHUNK // COMB


def _frame_of(c, p):
    return NCOMB * c + (p // COMB) + STRIDE * (p % COMB)


def _body(enc_hbm, dur_hbm, out_hbm, dur_v, csum_v, idx_v,
          bufs, gsems, osems):
    wid = lax.axis_index("s") * NC + lax.axis_index("c")
    b = wid // WPB
    q = wid % WPB
    fb = q * FPW
    out_base = b * ML + fb

    pltpu.sync_copy(dur_hbm.at[b], dur_v)

    carry = jnp.int32(0)
    for k in range(S // LANES):
        cs = plsc.cumsum(dur_v[pl.ds(k * LANES, LANES)]) + carry
        csum_v[pl.ds(k * LANES, LANES)] = cs
        carry = jnp.max(cs)
    total = carry

    total_local = jnp.clip(total - fb, 0, FPW)

    def idx_chunk(c):
        for v in range(VPC):
            f = (_frame_of(c, v * LANES)
                 + STRIDE * lax.iota(jnp.int32, LANES))
            j = fb + f
            r = jnp.zeros((LANES,), jnp.int32)
            for step in (256, 128, 64, 32, 16, 8, 4, 2, 1):
                cand = r + step
                vals = plsc.load_gather(csum_v, [cand - 1])
                r = jnp.where(vals <= j, cand, r)
            gidx = jnp.where(f < total_local, jnp.minimum(r, S - 1), f) + b * S
            idx_v[c, pl.ds(v * LANES, LANES)] = gidx

    def gather(c):
        return pltpu.async_copy(enc_hbm.at[idx_v.at[c]], bufs[c % NBUF],
                                gsems[c % NBUF])

    def finish_chunk(c, gd):
        gd.wait()
        buf = bufs[c % NBUF]
        obS = out_base // STRIDE

        def zrow(r, cc):
            for t in range(H // LANES):
                buf[r, pl.ds(t * LANES, LANES)] = jnp.zeros((LANES,),
                                                            jnp.float32)
            return cc

        ods = []
        for j in range(NCOMB):
            m = NCOMB * c + j
            pj = jnp.clip((total_local - m + STRIDE - 1) // STRIDE, 0, COMB)
            lax.fori_loop(j * COMB + pj, (j + 1) * COMB, zrow, 0)
            ods.append(pltpu.async_copy(
                buf.at[pl.ds(j * COMB, COMB)],
                out_hbm.at[pl.ds(obS, COMB), m],
                osems[c % NBUF]))
        return tuple(ods)

    gds = [None] * NCHUNK
    ods = [None] * NCHUNK
    for c in range(NCHUNK):
        idx_chunk(c)
        if c >= 2:
            ods[c - 2] = finish_chunk(c - 2, gds[c - 2])
        if c >= NBUF:
            for od in ods[c - NBUF]:
                od.wait()
        gds[c] = gather(c)
    for c in range(NCHUNK - 2, NCHUNK):
        ods[c] = finish_chunk(c, gds[c])
    for c in range(NCHUNK - NBUF, NCHUNK):
        for od in ods[c]:
            od.wait()


@jax.jit
def _expand(enc, dur):
    mesh = plsc.VectorSubcoreMesh(core_axis_name="c", subcore_axis_name="s",
                                  num_cores=NC, num_subcores=NS)
    return pl.kernel(
        _body,
        out_type=jax.ShapeDtypeStruct((B * ML // STRIDE, STRIDE, H), jnp.float32),
        mesh=mesh,
        compiler_params=pltpu.CompilerParams(needs_layout_passes=False),
        scratch_types=[
            pltpu.VMEM((S,), jnp.int32),
            pltpu.VMEM((S,), jnp.int32),
            pltpu.VMEM((NCHUNK, CHUNK), jnp.int32),
            [pltpu.VMEM((CHUNK, H), jnp.float32)] * NBUF,
            [pltpu.SemaphoreType.DMA] * NBUF,
            [pltpu.SemaphoreType.DMA] * NBUF,
        ],
    )(enc, dur)


def kernel(encoder_output, durations, max_length):
    enc = encoder_output.reshape(B * S, H)
    out = _expand(enc, durations)
    return out.reshape(B, ML, H)


# speedup vs baseline: 1.0724x; 1.0724x over previous
import jax
import jax.numpy as jnp
from jax import lax
from jax.experimental import pallas as pl
from jax.experimental.pallas import tpu as pltpu
from jax.experimental.pallas import tpu_sc as plsc

B = 8
S = 512
H = 256
ML = 2048
NC, NS = 2, 16
NW = NC * NS
WPB = NW // B
FPW = ML // WPB
CHUNK = 64
NCHUNK = FPW // CHUNK
NBUF = 4
LANES = 16
VPC = CHUNK // LANES
COMB = CHUNK // 2


def _frame_of(c, p):
    if p < COMB:
        return 2 * c + 16 * p
    return 2 * c + 1 + 16 * (p - COMB)


def _body(enc_hbm, dur_hbm, out_hbm, dur_v, csum_v, idx_v,
          bufs, gsems, osems):
    wid = lax.axis_index("s") * NC + lax.axis_index("c")
    b = wid // WPB
    q = wid % WPB
    fb = q * FPW
    out_base = b * ML + fb

    pltpu.sync_copy(dur_hbm.at[b], dur_v)

    carry = jnp.int32(0)
    for k in range(S // LANES):
        cs = plsc.cumsum(dur_v[pl.ds(k * LANES, LANES)]) + carry
        csum_v[pl.ds(k * LANES, LANES)] = cs
        carry = jnp.max(cs)
    total = carry

    total_local = jnp.clip(total - fb, 0, FPW)

    def idx_chunk(c):
        for v in range(VPC):
            f = (_frame_of(c, v * LANES)
                 + 16 * lax.iota(jnp.int32, LANES))
            j = fb + f
            r = jnp.zeros((LANES,), jnp.int32)
            for step in (256, 128, 64, 32, 16, 8, 4, 2, 1):
                cand = r + step
                vals = plsc.load_gather(csum_v, [cand - 1])
                r = jnp.where(vals <= j, cand, r)
            gidx = jnp.where(f < total_local, jnp.minimum(r, S - 1), f) + b * S
            idx_v[c, pl.ds(v * LANES, LANES)] = gidx

    def gather(c):
        return pltpu.async_copy(enc_hbm.at[idx_v.at[c]], bufs[c % NBUF],
                                gsems[c % NBUF])

    def finish_chunk(c, gd):
        gd.wait()
        buf = bufs[c % NBUF]
        p1 = jnp.clip((total_local - 2 * c + 15) // 16, 0, COMB)
        p2 = jnp.clip((total_local - 2 * c + 14) // 16, 0, COMB)

        def zrow(r, cc):
            for t in range(H // LANES):
                buf[r, pl.ds(t * LANES, LANES)] = jnp.zeros((LANES,),
                                                            jnp.float32)
            return cc

        lax.fori_loop(p1, COMB, zrow, 0)
        lax.fori_loop(COMB + p2, CHUNK, zrow, 0)
        ob16 = out_base // 16
        od0 = pltpu.async_copy(
            buf.at[pl.ds(0, COMB)],
            out_hbm.at[pl.ds(ob16, COMB), 2 * c],
            osems[c % NBUF])
        od1 = pltpu.async_copy(
            buf.at[pl.ds(COMB, COMB)],
            out_hbm.at[pl.ds(ob16, COMB), 2 * c + 1],
            osems[c % NBUF])
        return (od0, od1)

    gds = [None] * NCHUNK
    ods = [None] * NCHUNK
    for c in range(NCHUNK):
        idx_chunk(c)
        if c >= 2:
            ods[c - 2] = finish_chunk(c - 2, gds[c - 2])
        if c >= NBUF:
            for od in ods[c - NBUF]:
                od.wait()
        gds[c] = gather(c)
    for c in range(NCHUNK - 2, NCHUNK):
        ods[c] = finish_chunk(c, gds[c])
    for c in range(NCHUNK - NBUF, NCHUNK):
        for od in ods[c]:
            od.wait()


@jax.jit
def _expand(enc, dur):
    mesh = plsc.VectorSubcoreMesh(core_axis_name="c", subcore_axis_name="s",
                                  num_cores=NC, num_subcores=NS)
    return pl.kernel(
        _body,
        out_type=jax.ShapeDtypeStruct((B * ML // 16, 16, H), jnp.float32),
        mesh=mesh,
        compiler_params=pltpu.CompilerParams(needs_layout_passes=False),
        scratch_types=[
            pltpu.VMEM((S,), jnp.int32),
            pltpu.VMEM((S,), jnp.int32),
            pltpu.VMEM((NCHUNK, CHUNK), jnp.int32),
            [pltpu.VMEM((CHUNK, H), jnp.float32)] * NBUF,
            [pltpu.SemaphoreType.DMA] * NBUF,
            [pltpu.SemaphoreType.DMA] * NBUF,
        ],
    )(enc, dur)


def kernel(encoder_output, durations, max_length):
    enc = encoder_output.reshape(B * S, H)
    out = _expand(enc, durations)
    return out.reshape(B, ML, H)


# speedup vs baseline: 1.0816x; 1.0086x over previous
import jax
import jax.numpy as jnp
from jax import lax
from jax.experimental import pallas as pl
from jax.experimental.pallas import tpu as pltpu
from jax.experimental.pallas import tpu_sc as plsc

B = 8
S = 512
H = 256
ML = 2048
NC, NS = 2, 16
NW = NC * NS
WPB = NW // B
FPW = ML // WPB
CHUNK = 64
NCHUNK = FPW // CHUNK
NBUF = 4
LANES = 16
VPC = CHUNK // LANES
COMB = CHUNK // 2


def _frame_of(c, p):
    if p < COMB:
        return 2 * c + 16 * p
    return 2 * c + 1 + 16 * (p - COMB)


def _body(enc_hbm, dur_hbm, out_hbm, dur_v, csum_v, idx_v,
          bufs, gsems, osems):
    wid = lax.axis_index("s") * NC + lax.axis_index("c")
    b = wid // WPB
    q = wid % WPB
    fb = q * FPW
    out_base = b * ML + fb

    pltpu.sync_copy(dur_hbm.at[b], dur_v)

    carry = jnp.int32(0)
    for k in range(S // LANES):
        cs = plsc.cumsum(dur_v[pl.ds(k * LANES, LANES)]) + carry
        csum_v[pl.ds(k * LANES, LANES)] = cs
        carry = jnp.max(cs)
    total = carry

    total_local = jnp.clip(total - fb, 0, FPW)

    def idx_chunk(c):
        for v in range(VPC):
            f = (_frame_of(c, v * LANES)
                 + 16 * lax.iota(jnp.int32, LANES))
            j = fb + f
            r = jnp.zeros((LANES,), jnp.int32)
            for step in (256, 128, 64, 32, 16, 8, 4, 2, 1):
                cand = r + step
                vals = plsc.load_gather(csum_v, [cand - 1])
                r = jnp.where(vals <= j, cand, r)
            gidx = jnp.where(f < total_local, jnp.minimum(r, S - 1), f) + b * S
            idx_v[c, pl.ds(v * LANES, LANES)] = gidx

    nbuf_of = {c: i % NBUF for i, c in enumerate([0, 4, 1, 5, 2, 6, 3, 7])}

    def gather(c):
        return pltpu.async_copy(enc_hbm.at[idx_v.at[c]], bufs[nbuf_of[c]],
                                gsems[nbuf_of[c]])

    def finish_chunk(c, gd):
        gd.wait()
        buf = bufs[nbuf_of[c]]
        p1 = jnp.clip((total_local - 2 * c + 15) // 16, 0, COMB)
        p2 = jnp.clip((total_local - 2 * c + 14) // 16, 0, COMB)

        def zrow(r, cc):
            for t in range(H // LANES):
                buf[r, pl.ds(t * LANES, LANES)] = jnp.zeros((LANES,),
                                                            jnp.float32)
            return cc

        lax.fori_loop(p1, COMB, zrow, 0)
        lax.fori_loop(COMB + p2, CHUNK, zrow, 0)
        ob16 = out_base // 16
        od0 = pltpu.async_copy(
            buf.at[pl.ds(0, COMB)],
            out_hbm.at[pl.ds(ob16, COMB), 2 * c],
            osems[nbuf_of[c]])
        od1 = pltpu.async_copy(
            buf.at[pl.ds(COMB, COMB)],
            out_hbm.at[pl.ds(ob16, COMB), 2 * c + 1],
            osems[nbuf_of[c]])
        return (od0, od1)

    order = [0, 4, 1, 5, 2, 6, 3, 7]
    gds = [None] * NCHUNK
    ods = [None] * NCHUNK
    for ci in range(NCHUNK):
        idx_chunk(order[ci])
        if ci >= 2:
            cc = order[ci - 2]
            ods[ci - 2] = finish_chunk(cc, gds[ci - 2])
        if ci >= NBUF:
            for od in ods[ci - NBUF]:
                od.wait()
        gds[ci] = gather(order[ci])
    for ci in range(NCHUNK - 2, NCHUNK):
        ods[ci] = finish_chunk(order[ci], gds[ci])
    for ci in range(NCHUNK - NBUF, NCHUNK):
        for od in ods[ci]:
            od.wait()


@jax.jit
def _expand(enc, dur):
    mesh = plsc.VectorSubcoreMesh(core_axis_name="c", subcore_axis_name="s",
                                  num_cores=NC, num_subcores=NS)
    return pl.kernel(
        _body,
        out_type=jax.ShapeDtypeStruct((B * ML // 16, 16, H), jnp.float32),
        mesh=mesh,
        compiler_params=pltpu.CompilerParams(needs_layout_passes=False),
        scratch_types=[
            pltpu.VMEM((S,), jnp.int32),
            pltpu.VMEM((S,), jnp.int32),
            pltpu.VMEM((NCHUNK, CHUNK), jnp.int32),
            [pltpu.VMEM((CHUNK, H), jnp.float32)] * NBUF,
            [pltpu.SemaphoreType.DMA] * NBUF,
            [pltpu.SemaphoreType.DMA] * NBUF,
        ],
    )(enc, dur)


def kernel(encoder_output, durations, max_length):
    enc = encoder_output.reshape(B * S, H)
    out = _expand(enc, durations)
    return out.reshape(B, ML, H)
